# baseline (speedup 1.0000x reference)
import functools

import jax
import jax.numpy as jnp
from jax import lax
from jax.experimental import pallas as pl
from jax.experimental.pallas import tpu as pltpu
from jax.experimental.pallas import tpu_sc as plsc

NC = 2
NS = 16
NW = NC * NS
CHUNK = 128


@functools.partial(jax.jit, static_argnames=("nchunk", "embed"))
def _sc_gather(xw, table, nchunk, embed):
    mesh = plsc.VectorSubcoreMesh(core_axis_name="c", subcore_axis_name="s")

    @functools.partial(
        pl.kernel,
        out_type=jax.ShapeDtypeStruct((NW, nchunk, CHUNK, embed), jnp.float32),
        mesh=mesh,
        scratch_types=[
            pltpu.VMEM((nchunk, CHUNK), jnp.int32),
            pltpu.VMEM((CHUNK, embed), jnp.float32),
            pltpu.SemaphoreType.DMA,
        ],
        compiler_params=pltpu.CompilerParams(use_tc_tiling_on_sc=False),
    )
    def k(x_hbm, table_hbm, out_hbm, idx_v, rows_v, sem):
        wid = lax.axis_index("s") * NC + lax.axis_index("c")
        pltpu.sync_copy(x_hbm.at[wid], idx_v)

        def body(j, carry):
            pltpu.async_copy(table_hbm.at[idx_v.at[j]], rows_v, sem).wait()
            pltpu.sync_copy(rows_v, out_hbm.at[wid, j])
            return carry

        lax.fori_loop(0, nchunk, body, 0)

    return k(xw, table)


def kernel(x, table):
    batch, tlen = x.shape
    embed = table.shape[1]
    total = batch * tlen
    assert total % (NW * CHUNK) == 0
    nchunk = total // (NW * CHUNK)
    xw = x.astype(jnp.int32).reshape(NW, nchunk, CHUNK)
    out = _sc_gather(xw, table, nchunk, embed)
    return out.reshape(batch, tlen, embed)


# speedup vs baseline: 1.1127x; 1.1127x over previous
import functools

import jax
import jax.numpy as jnp
from jax import lax
from jax.experimental import pallas as pl
from jax.experimental.pallas import tpu as pltpu
from jax.experimental.pallas import tpu_sc as plsc

NC = 2
NS = 16
NW = NC * NS
CHUNK = 128
NBUF = 8


@functools.partial(jax.jit, static_argnames=("nchunk", "embed"))
def _sc_gather(xw, table, nchunk, embed):
    mesh = plsc.VectorSubcoreMesh(core_axis_name="c", subcore_axis_name="s")
    ngroups = nchunk // NBUF

    @functools.partial(
        pl.kernel,
        out_type=jax.ShapeDtypeStruct((NW, nchunk, CHUNK, embed), jnp.float32),
        mesh=mesh,
        scratch_types=[
            pltpu.VMEM((nchunk, CHUNK), jnp.int32),
            pltpu.VMEM((NBUF, CHUNK, embed), jnp.float32),
        ] + [pltpu.SemaphoreType.DMA] * (2 * NBUF),
        compiler_params=pltpu.CompilerParams(use_tc_tiling_on_sc=False),
    )
    def k(x_hbm, table_hbm, out_hbm, idx_v, rows_v, *sems):
        gsem = sems[:NBUF]
        ssem = sems[NBUF:]
        wid = lax.axis_index("s") * NC + lax.axis_index("c")
        pltpu.sync_copy(x_hbm.at[wid], idx_v)

        def start_gather(b, j):
            pltpu.async_copy(table_hbm.at[idx_v.at[j]], rows_v.at[b], gsem[b])

        def wait_gather(b, j):
            pltpu.make_async_copy(
                table_hbm.at[idx_v.at[j]], rows_v.at[b], gsem[b]).wait()

        def start_store(b, j):
            pltpu.async_copy(rows_v.at[b], out_hbm.at[wid, j], ssem[b])

        def wait_store(b, j):
            pltpu.make_async_copy(
                rows_v.at[b], out_hbm.at[wid, j], ssem[b]).wait()

        for b in range(NBUF):
            start_gather(b, b)

        def body(g, carry):
            for b in range(NBUF):
                j = g * NBUF + b
                wait_gather(b, j)
                start_store(b, j)
            for b in range(NBUF):
                j = g * NBUF + b
                wait_store(b, j)
                start_gather(b, j + NBUF)
            return carry

        lax.fori_loop(0, ngroups - 1, body, 0)

        g = ngroups - 1
        for b in range(NBUF):
            j = g * NBUF + b
            wait_gather(b, j)
            start_store(b, j)
        for b in range(NBUF):
            wait_store(b, g * NBUF + b)

    return k(xw, table)


def kernel(x, table):
    batch, tlen = x.shape
    embed = table.shape[1]
    total = batch * tlen
    assert total % (NW * CHUNK) == 0
    nchunk = total // (NW * CHUNK)
    assert nchunk % NBUF == 0
    xw = x.astype(jnp.int32).reshape(NW, nchunk, CHUNK)
    out = _sc_gather(xw, table, nchunk, embed)
    return out.reshape(batch, tlen, embed)
